# baseline (speedup 1.0000x reference)
import jax
import jax.numpy as jnp
from jax.experimental import pallas as pl

S = 2048
D = 1024
NH = 16
DH = 64
WIN = 512
NPOOL = 192
POOLED = NPOOL * 8
TK = 64
NEG = -1000000000.0
SCALE = 1.0 / 8.0
F32 = jnp.float32
BF16 = jnp.bfloat16

QBLK = 256
CQBLK = 512
PBLK = 256


def _proj_kernel(xf_ref, xb_ref, wqkv_ref, wcq_ref, wtq_ref, wimp_ref, bqkv_ref,
                 bcq_ref, btq_ref, bimp_ref,
                 qkv_ref, cq_ref, tq_ref, imp_ref):
    xb = xb_ref[...]
    qkv_ref[...] = (
        jnp.dot(xb, wqkv_ref[...], preferred_element_type=F32) + bqkv_ref[...]
    ).astype(BF16)
    cq_ref[...] = (
        jnp.dot(xb, wcq_ref[...], preferred_element_type=F32) + bcq_ref[...]
    ).astype(BF16)
    tq_ref[...] = (
        jnp.dot(xb, wtq_ref[...], preferred_element_type=F32) + btq_ref[...]
    ).astype(BF16)
    imp_ref[...] = (
        jnp.dot(xf_ref[...], wimp_ref[...], preferred_element_type=F32)
        + bimp_ref[...]
    )


def _pool_ckcv_kernel(a_ref, x_ref, w_ref, b_ref, o_ref):
    comp = jnp.dot(a_ref[...], x_ref[...], preferred_element_type=F32)
    o_ref[...] = (
        jnp.dot(comp.astype(BF16), w_ref[...], preferred_element_type=F32)
        + b_ref[...]
    ).astype(BF16)


def _rank_kernel(impc_ref, impr_ref, o_ref):
    p = pl.program_id(0)
    ic = impc_ref[...]
    ir = impr_ref[...]
    gt = (ir > ic).astype(F32)
    jj = jax.lax.broadcasted_iota(jnp.int32, (QBLK, S), 1)
    ii = p * QBLK + jax.lax.broadcasted_iota(jnp.int32, (QBLK, S), 0)
    eq = ((ir == ic) & (jj < ii)).astype(F32)
    o_ref[...] = jnp.sum(gt + eq, axis=1, keepdims=True)


def _sel_kernel(rank_ref, x_ref, w_ref, b_ref, tkv_ref, tidx_ref):
    r = rank_ref[...]
    rows = jax.lax.broadcasted_iota(jnp.int32, (TK, S), 0).astype(F32)
    P = (r == rows).astype(F32)
    cols = jax.lax.broadcasted_iota(jnp.int32, (TK, S), 1).astype(F32)
    tidx_ref[...] = jnp.sum(P * cols, axis=1, keepdims=True)
    sel = jnp.dot(P, x_ref[...], preferred_element_type=F32)
    tkv_ref[...] = (
        jnp.dot(sel, w_ref[...], preferred_element_type=F32) + b_ref[...]
    ).astype(BF16)


def _softmax(s):
    m = jnp.max(s, axis=-1, keepdims=True)
    e = jnp.exp(s - m)
    return e / jnp.sum(e, axis=-1, keepdims=True)


def _local_attn_kernel(q_ref, k0_ref, k1_ref, k2_ref, v0_ref, v1_ref, v2_ref, o_ref):
    i = pl.program_id(1)
    q = q_ref[...]
    krefs = (k0_ref, k1_ref, k2_ref)
    vrefs = (v0_ref, v1_ref, v2_ref)
    qabs = i * QBLK + jax.lax.broadcasted_iota(jnp.int32, (QBLK, QBLK), 0)
    kiota = jax.lax.broadcasted_iota(jnp.int32, (QBLK, QBLK), 1)
    halves = []
    for h in range(2):
        qa = q[:, h * DH:(h + 1) * DH] * jnp.asarray(SCALE, BF16)
        schunks = []
        for c in range(3):
            kb = krefs[c][...][:, h * DH:(h + 1) * DH]
            s = jax.lax.dot_general(
                qa, kb, (((1,), (1,)), ((), ())), preferred_element_type=F32
            )
            boff = i - 2 + c
            kabs = jnp.maximum(boff, 0) * QBLK + kiota
            diff = qabs - kabs
            valid = (boff >= 0) & (diff >= 0) & (diff < WIN)
            schunks.append(jnp.where(valid, s, NEG))
        sfull = jnp.concatenate(schunks, axis=1)
        p = _softmax(sfull).astype(BF16)
        vfull = jnp.concatenate(
            [vr[...][:, h * DH:(h + 1) * DH] for vr in vrefs], axis=0
        )
        halves.append(jnp.dot(p, vfull, preferred_element_type=F32))
    o_ref[...] = jnp.concatenate(halves, axis=1).astype(BF16)


def _ctattn_kernel(cq_ref, tq_ref, ck_ref, cv_ref, tk_ref, tv_ref, tpos_ref,
                   co_ref, to_ref):
    iq = pl.program_id(1)
    cq = cq_ref[...]
    tq = tq_ref[...]
    qi_c = iq * CQBLK + jax.lax.broadcasted_iota(jnp.int32, (CQBLK, NPOOL), 0)
    pend = (jax.lax.broadcasted_iota(jnp.int32, (CQBLK, NPOOL), 1) + 1) * 8
    cmask = qi_c >= pend
    qi_t = iq * CQBLK + jax.lax.broadcasted_iota(jnp.int32, (CQBLK, TK), 0)
    tmask = qi_t.astype(F32) >= tpos_ref[...]
    chalves, thalves = [], []
    sc = jnp.asarray(SCALE, BF16)
    for h in range(2):
        sl = slice(h * DH, (h + 1) * DH)
        s = jax.lax.dot_general(
            cq[:, sl] * sc, ck_ref[...][:, sl], (((1,), (1,)), ((), ())),
            preferred_element_type=F32,
        )
        p = _softmax(jnp.where(cmask, s, NEG)).astype(BF16)
        chalves.append(jnp.dot(p, cv_ref[...][:, sl], preferred_element_type=F32))
        st = jax.lax.dot_general(
            tq[:, sl] * sc, tk_ref[...][:, sl], (((1,), (1,)), ((), ())),
            preferred_element_type=F32,
        )
        pt = _softmax(jnp.where(tmask, st, NEG)).astype(BF16)
        thalves.append(jnp.dot(pt, tv_ref[...][:, sl], preferred_element_type=F32))
    co_ref[...] = jnp.concatenate(chalves, axis=1).astype(BF16)
    to_ref[...] = jnp.concatenate(thalves, axis=1).astype(BF16)


def _combine_kernel(x_ref, loc_ref, co_ref, to_ref,
                    wl_ref, bl_ref, wc_ref, bc_ref, wt_ref, bt_ref,
                    wgc_ref, bgc_ref, wgt_ref, bgt_ref, o_ref):
    xb = x_ref[...]
    out = jnp.dot(loc_ref[...], wl_ref[...], preferred_element_type=F32) + bl_ref[...]
    gc = jax.nn.sigmoid(
        jnp.dot(xb, wgc_ref[...], preferred_element_type=F32) + bgc_ref[...]
    )
    cproj = jnp.dot(co_ref[...], wc_ref[...], preferred_element_type=F32) + bc_ref[...]
    out += gc * cproj
    gt = jax.nn.sigmoid(
        jnp.dot(xb, wgt_ref[...], preferred_element_type=F32) + bgt_ref[...]
    )
    tproj = jnp.dot(to_ref[...], wt_ref[...], preferred_element_type=F32) + bt_ref[...]
    out += gt * tproj
    o_ref[...] = out


def kernel(x, W_qkv, b_qkv, W_lout, b_lout, W_cq, b_cq, W_ck, b_ck, W_cv, b_cv,
           W_cout, b_cout, W_gc, b_gc, W_imp, b_imp, W_tq, b_tq, W_tk, b_tk,
           W_tv, b_tv, W_tout, b_tout, W_gt, b_gt):
    x2 = x.reshape(S, D)
    xb = x2.astype(BF16)

    qkv, cq, tq, imp = pl.pallas_call(
        _proj_kernel,
        grid=(S // PBLK,),
        in_specs=[
            pl.BlockSpec((PBLK, D), lambda m: (m, 0)),
            pl.BlockSpec((PBLK, D), lambda m: (m, 0)),
            pl.BlockSpec((D, 3 * D), lambda m: (0, 0)),
            pl.BlockSpec((D, D), lambda m: (0, 0)),
            pl.BlockSpec((D, D), lambda m: (0, 0)),
            pl.BlockSpec((D, 1), lambda m: (0, 0)),
            pl.BlockSpec((1, 3 * D), lambda m: (0, 0)),
            pl.BlockSpec((1, D), lambda m: (0, 0)),
            pl.BlockSpec((1, D), lambda m: (0, 0)),
            pl.BlockSpec((1, 1), lambda m: (0, 0)),
        ],
        out_specs=[
            pl.BlockSpec((PBLK, 3 * D), lambda m: (m, 0)),
            pl.BlockSpec((PBLK, D), lambda m: (m, 0)),
            pl.BlockSpec((PBLK, D), lambda m: (m, 0)),
            pl.BlockSpec((PBLK, 1), lambda m: (m, 0)),
        ],
        out_shape=[
            jax.ShapeDtypeStruct((S, 3 * D), BF16),
            jax.ShapeDtypeStruct((S, D), BF16),
            jax.ShapeDtypeStruct((S, D), BF16),
            jax.ShapeDtypeStruct((S, 1), F32),
        ],
    )(x2, xb, W_qkv.astype(BF16), W_cq.astype(BF16), W_tq.astype(BF16), W_imp,
      b_qkv.reshape(1, -1), b_cq.reshape(1, -1), b_tq.reshape(1, -1),
      b_imp.reshape(1, 1))

    pool_a = (jnp.repeat(jnp.eye(NPOOL, dtype=F32), 8, axis=1) * 0.125).astype(BF16)
    w_ckv = jnp.concatenate([W_ck, W_cv], axis=1)
    b_ckv = jnp.concatenate([b_ck, b_cv]).reshape(1, -1)
    ckcv = pl.pallas_call(
        _pool_ckcv_kernel,
        grid=(1,),
        in_specs=[
            pl.BlockSpec((NPOOL, POOLED), lambda g: (0, 0)),
            pl.BlockSpec((POOLED, D), lambda g: (0, 0)),
            pl.BlockSpec((D, 2 * D), lambda g: (0, 0)),
            pl.BlockSpec((1, 2 * D), lambda g: (0, 0)),
        ],
        out_specs=pl.BlockSpec((NPOOL, 2 * D), lambda g: (0, 0)),
        out_shape=jax.ShapeDtypeStruct((NPOOL, 2 * D), BF16),
    )(pool_a, xb, w_ckv.astype(BF16), b_ckv)

    rank = pl.pallas_call(
        _rank_kernel,
        grid=(S // QBLK,),
        in_specs=[
            pl.BlockSpec((QBLK, 1), lambda p: (p, 0)),
            pl.BlockSpec((1, S), lambda p: (0, 0)),
        ],
        out_specs=pl.BlockSpec((QBLK, 1), lambda p: (p, 0)),
        out_shape=jax.ShapeDtypeStruct((S, 1), F32),
    )(imp, imp.reshape(1, S))

    w_tkv = jnp.concatenate([W_tk, W_tv], axis=1)
    b_tkv = jnp.concatenate([b_tk, b_tv]).reshape(1, -1)
    tkv, tidx = pl.pallas_call(
        _sel_kernel,
        grid=(1,),
        in_specs=[
            pl.BlockSpec((1, S), lambda g: (0, 0)),
            pl.BlockSpec((S, D), lambda g: (0, 0)),
            pl.BlockSpec((D, 2 * D), lambda g: (0, 0)),
            pl.BlockSpec((1, 2 * D), lambda g: (0, 0)),
        ],
        out_specs=[
            pl.BlockSpec((TK, 2 * D), lambda g: (0, 0)),
            pl.BlockSpec((TK, 1), lambda g: (0, 0)),
        ],
        out_shape=[
            jax.ShapeDtypeStruct((TK, 2 * D), BF16),
            jax.ShapeDtypeStruct((TK, 1), F32),
        ],
    )(rank.reshape(1, S), x2, w_tkv, b_tkv)

    nhp = NH // 2
    local = pl.pallas_call(
        _local_attn_kernel,
        grid=(nhp, S // QBLK),
        in_specs=[
            pl.BlockSpec((QBLK, 128), lambda hp, i: (i, hp)),
            pl.BlockSpec((QBLK, 128), lambda hp, i: (jnp.maximum(i - 2, 0), 8 + hp)),
            pl.BlockSpec((QBLK, 128), lambda hp, i: (jnp.maximum(i - 1, 0), 8 + hp)),
            pl.BlockSpec((QBLK, 128), lambda hp, i: (i, 8 + hp)),
            pl.BlockSpec((QBLK, 128), lambda hp, i: (jnp.maximum(i - 2, 0), 16 + hp)),
            pl.BlockSpec((QBLK, 128), lambda hp, i: (jnp.maximum(i - 1, 0), 16 + hp)),
            pl.BlockSpec((QBLK, 128), lambda hp, i: (i, 16 + hp)),
        ],
        out_specs=pl.BlockSpec((QBLK, 128), lambda hp, i: (i, hp)),
        out_shape=jax.ShapeDtypeStruct((S, D), BF16),
    )(qkv, qkv, qkv, qkv, qkv, qkv, qkv)

    cout, tout = pl.pallas_call(
        _ctattn_kernel,
        grid=(nhp, S // CQBLK),
        in_specs=[
            pl.BlockSpec((CQBLK, 128), lambda hp, i: (i, hp)),
            pl.BlockSpec((CQBLK, 128), lambda hp, i: (i, hp)),
            pl.BlockSpec((NPOOL, 128), lambda hp, i: (0, hp)),
            pl.BlockSpec((NPOOL, 128), lambda hp, i: (0, 8 + hp)),
            pl.BlockSpec((TK, 128), lambda hp, i: (0, hp)),
            pl.BlockSpec((TK, 128), lambda hp, i: (0, 8 + hp)),
            pl.BlockSpec((1, TK), lambda hp, i: (0, 0)),
        ],
        out_specs=[
            pl.BlockSpec((CQBLK, 128), lambda hp, i: (i, hp)),
            pl.BlockSpec((CQBLK, 128), lambda hp, i: (i, hp)),
        ],
        out_shape=[
            jax.ShapeDtypeStruct((S, D), BF16),
            jax.ShapeDtypeStruct((S, D), BF16),
        ],
    )(cq, tq, ckcv, ckcv, tkv, tkv, tidx.reshape(1, TK))

    out = pl.pallas_call(
        _combine_kernel,
        grid=(S // PBLK,),
        in_specs=[
            pl.BlockSpec((PBLK, D), lambda m: (m, 0)),
            pl.BlockSpec((PBLK, D), lambda m: (m, 0)),
            pl.BlockSpec((PBLK, D), lambda m: (m, 0)),
            pl.BlockSpec((PBLK, D), lambda m: (m, 0)),
            pl.BlockSpec((D, D), lambda m: (0, 0)),
            pl.BlockSpec((1, D), lambda m: (0, 0)),
            pl.BlockSpec((D, D), lambda m: (0, 0)),
            pl.BlockSpec((1, D), lambda m: (0, 0)),
            pl.BlockSpec((D, D), lambda m: (0, 0)),
            pl.BlockSpec((1, D), lambda m: (0, 0)),
            pl.BlockSpec((D, D), lambda m: (0, 0)),
            pl.BlockSpec((1, D), lambda m: (0, 0)),
            pl.BlockSpec((D, D), lambda m: (0, 0)),
            pl.BlockSpec((1, D), lambda m: (0, 0)),
        ],
        out_specs=pl.BlockSpec((PBLK, D), lambda m: (m, 0)),
        out_shape=jax.ShapeDtypeStruct((S, D), F32),
    )(xb, local, cout, tout,
      W_lout.astype(BF16), b_lout.reshape(1, -1),
      W_cout.astype(BF16), b_cout.reshape(1, -1),
      W_tout.astype(BF16), b_tout.reshape(1, -1),
      W_gc.astype(BF16), b_gc.reshape(1, -1),
      W_gt.astype(BF16), b_gt.reshape(1, -1))

    return out.reshape(1, S, D)


# speedup vs baseline: 1.3524x; 1.3524x over previous
import jax
import jax.numpy as jnp
from jax.experimental import pallas as pl

S = 2048
D = 1024
NH = 16
DH = 64
WIN = 512
NPOOL = 192
POOLED = NPOOL * 8
TK = 64
NEG = -1000000000.0
SCALE = 1.0 / 8.0
F32 = jnp.float32
BF16 = jnp.bfloat16

QBLK = 256
CQBLK = 512
PBLK = 256


def _proj_kernel(x_ref, wqkv_ref, wcq_ref, wtq_ref, wimp_ref, bqkv_ref,
                 bcq_ref, btq_ref, bimp_ref,
                 qkv_ref, cq_ref, tq_ref, imp_ref):
    xb = x_ref[...]
    qkv_ref[...] = (
        jnp.dot(xb, wqkv_ref[...], preferred_element_type=F32) + bqkv_ref[...]
    ).astype(BF16)
    cq_ref[...] = (
        jnp.dot(xb, wcq_ref[...], preferred_element_type=F32) + bcq_ref[...]
    ).astype(BF16)
    tq_ref[...] = (
        jnp.dot(xb, wtq_ref[...], preferred_element_type=F32) + btq_ref[...]
    ).astype(BF16)
    imp_ref[...] = (
        jnp.dot(xb, wimp_ref[...], preferred_element_type=F32) + bimp_ref[...]
    )


def _pool_ckcv_kernel(a_ref, x_ref, w_ref, b_ref, o_ref):
    comp = jnp.dot(a_ref[...], x_ref[...], preferred_element_type=F32)
    o_ref[...] = (
        jnp.dot(comp, w_ref[...], preferred_element_type=F32) + b_ref[...]
    ).astype(BF16)


def _rank_kernel(impc_ref, impr_ref, o_ref):
    p = pl.program_id(0)
    ic = impc_ref[...]
    ir = impr_ref[...]
    gt = (ir > ic).astype(F32)
    jj = jax.lax.broadcasted_iota(jnp.int32, (QBLK, S), 1)
    ii = p * QBLK + jax.lax.broadcasted_iota(jnp.int32, (QBLK, S), 0)
    eq = ((ir == ic) & (jj < ii)).astype(F32)
    o_ref[...] = jnp.sum(gt + eq, axis=1, keepdims=True)


def _sel_kernel(rank_ref, x_ref, w_ref, b_ref, tkv_ref, tidx_ref):
    r = rank_ref[...]
    rows = jax.lax.broadcasted_iota(jnp.int32, (TK, S), 0).astype(F32)
    P = (r == rows).astype(F32)
    cols = jax.lax.broadcasted_iota(jnp.int32, (TK, S), 1).astype(F32)
    tidx_ref[...] = jnp.sum(P * cols, axis=1, keepdims=True)
    sel = jnp.dot(P, x_ref[...], preferred_element_type=F32)
    tkv_ref[...] = (
        jnp.dot(sel, w_ref[...], preferred_element_type=F32) + b_ref[...]
    ).astype(BF16)


def _softmax(s):
    m = jnp.max(s, axis=-1, keepdims=True)
    e = jnp.exp(s - m)
    return e / jnp.sum(e, axis=-1, keepdims=True)


def _local_attn_kernel(q_ref, k0_ref, k1_ref, k2_ref, v0_ref, v1_ref, v2_ref, o_ref):
    i = pl.program_id(1)
    q = q_ref[...]
    krefs = (k0_ref, k1_ref, k2_ref)
    vrefs = (v0_ref, v1_ref, v2_ref)
    base = (jax.lax.broadcasted_iota(jnp.int32, (QBLK, QBLK), 0)
            - jax.lax.broadcasted_iota(jnp.int32, (QBLK, QBLK), 1))
    bias0 = jnp.where((i >= 2) & (base + 2 * QBLK < WIN), 0.0, NEG)
    bias1 = jnp.where(i >= 1, 0.0, NEG)
    bias2 = jnp.where(base >= 0, 0.0, NEG)
    halves = []
    for h in range(2):
        qa = q[:, h * DH:(h + 1) * DH] * jnp.asarray(SCALE, BF16)
        kb0 = krefs[0][...][:, h * DH:(h + 1) * DH]
        kb1 = krefs[1][...][:, h * DH:(h + 1) * DH]
        kb2 = krefs[2][...][:, h * DH:(h + 1) * DH]
        dn = (((1,), (1,)), ((), ()))
        e0 = jnp.exp(jax.lax.dot_general(qa, kb0, dn, preferred_element_type=F32)
                     + bias0)
        e1 = jnp.exp(jax.lax.dot_general(qa, kb1, dn, preferred_element_type=F32)
                     + bias1)
        e2 = jnp.exp(jax.lax.dot_general(qa, kb2, dn, preferred_element_type=F32)
                     + bias2)
        denom = (jnp.sum(e0, axis=1, keepdims=True)
                 + jnp.sum(e1, axis=1, keepdims=True)
                 + jnp.sum(e2, axis=1, keepdims=True))
        acc = jnp.dot(e0, vrefs[0][...][:, h * DH:(h + 1) * DH],
                      preferred_element_type=F32)
        acc += jnp.dot(e1, vrefs[1][...][:, h * DH:(h + 1) * DH],
                       preferred_element_type=F32)
        acc += jnp.dot(e2, vrefs[2][...][:, h * DH:(h + 1) * DH],
                       preferred_element_type=F32)
        halves.append(acc / denom)
    o_ref[...] = jnp.concatenate(halves, axis=1).astype(BF16)


def _ctattn_kernel(cq_ref, tq_ref, ck_ref, cv_ref, tk_ref, tv_ref, tpos_ref,
                   co_ref, to_ref):
    iq = pl.program_id(1)
    cq = cq_ref[...]
    tq = tq_ref[...]
    qi_c = iq * CQBLK + jax.lax.broadcasted_iota(jnp.int32, (CQBLK, NPOOL), 0)
    pend = (jax.lax.broadcasted_iota(jnp.int32, (CQBLK, NPOOL), 1) + 1) * 8
    cmask = qi_c >= pend
    qi_t = iq * CQBLK + jax.lax.broadcasted_iota(jnp.int32, (CQBLK, TK), 0)
    tmask = qi_t.astype(F32) >= tpos_ref[...]
    chalves, thalves = [], []
    sc = jnp.asarray(SCALE, BF16)
    dn = (((1,), (1,)), ((), ()))
    for h in range(2):
        sl = slice(h * DH, (h + 1) * DH)
        s = jnp.where(cmask, jax.lax.dot_general(
            cq[:, sl] * sc, ck_ref[...][:, sl], dn, preferred_element_type=F32
        ), NEG)
        e = jnp.exp(s - jnp.max(s, axis=1, keepdims=True))
        num = jnp.dot(e, cv_ref[...][:, sl], preferred_element_type=F32)
        chalves.append(num / jnp.sum(e, axis=1, keepdims=True))
        st = jnp.where(tmask, jax.lax.dot_general(
            tq[:, sl] * sc, tk_ref[...][:, sl], dn, preferred_element_type=F32
        ), NEG)
        et = jnp.exp(st - jnp.max(st, axis=1, keepdims=True))
        numt = jnp.dot(et, tv_ref[...][:, sl], preferred_element_type=F32)
        thalves.append(numt / jnp.sum(et, axis=1, keepdims=True))
    co_ref[...] = jnp.concatenate(chalves, axis=1).astype(BF16)
    to_ref[...] = jnp.concatenate(thalves, axis=1).astype(BF16)


def _combine_kernel(x_ref, loc_ref, co_ref, to_ref,
                    wl_ref, bl_ref, wc_ref, bc_ref, wt_ref, bt_ref,
                    wgc_ref, bgc_ref, wgt_ref, bgt_ref, o_ref):
    xb = x_ref[...]
    out = jnp.dot(loc_ref[...], wl_ref[...], preferred_element_type=F32) + bl_ref[...]
    gc = jax.nn.sigmoid(
        jnp.dot(xb, wgc_ref[...], preferred_element_type=F32) + bgc_ref[...]
    )
    cproj = jnp.dot(co_ref[...], wc_ref[...], preferred_element_type=F32) + bc_ref[...]
    out += gc * cproj
    gt = jax.nn.sigmoid(
        jnp.dot(xb, wgt_ref[...], preferred_element_type=F32) + bgt_ref[...]
    )
    tproj = jnp.dot(to_ref[...], wt_ref[...], preferred_element_type=F32) + bt_ref[...]
    out += gt * tproj
    o_ref[...] = out


def kernel(x, W_qkv, b_qkv, W_lout, b_lout, W_cq, b_cq, W_ck, b_ck, W_cv, b_cv,
           W_cout, b_cout, W_gc, b_gc, W_imp, b_imp, W_tq, b_tq, W_tk, b_tk,
           W_tv, b_tv, W_tout, b_tout, W_gt, b_gt):
    x2 = x.reshape(S, D)

    qkv, cq, tq, imp = pl.pallas_call(
        _proj_kernel,
        grid=(S // PBLK,),
        in_specs=[
            pl.BlockSpec((PBLK, D), lambda m: (m, 0)),
            pl.BlockSpec((D, 3 * D), lambda m: (0, 0)),
            pl.BlockSpec((D, D), lambda m: (0, 0)),
            pl.BlockSpec((D, D), lambda m: (0, 0)),
            pl.BlockSpec((D, 1), lambda m: (0, 0)),
            pl.BlockSpec((1, 3 * D), lambda m: (0, 0)),
            pl.BlockSpec((1, D), lambda m: (0, 0)),
            pl.BlockSpec((1, D), lambda m: (0, 0)),
            pl.BlockSpec((1, 1), lambda m: (0, 0)),
        ],
        out_specs=[
            pl.BlockSpec((PBLK, 3 * D), lambda m: (m, 0)),
            pl.BlockSpec((PBLK, D), lambda m: (m, 0)),
            pl.BlockSpec((PBLK, D), lambda m: (m, 0)),
            pl.BlockSpec((PBLK, 1), lambda m: (m, 0)),
        ],
        out_shape=[
            jax.ShapeDtypeStruct((S, 3 * D), BF16),
            jax.ShapeDtypeStruct((S, D), BF16),
            jax.ShapeDtypeStruct((S, D), BF16),
            jax.ShapeDtypeStruct((S, 1), F32),
        ],
    )(x2, W_qkv, W_cq, W_tq, W_imp,
      b_qkv.reshape(1, -1), b_cq.reshape(1, -1), b_tq.reshape(1, -1),
      b_imp.reshape(1, 1))

    pool_a = jnp.repeat(jnp.eye(NPOOL, dtype=F32), 8, axis=1) * 0.125
    w_ckv = jnp.concatenate([W_ck, W_cv], axis=1)
    b_ckv = jnp.concatenate([b_ck, b_cv]).reshape(1, -1)
    ckcv = pl.pallas_call(
        _pool_ckcv_kernel,
        grid=(1,),
        in_specs=[
            pl.BlockSpec((NPOOL, POOLED), lambda g: (0, 0)),
            pl.BlockSpec((POOLED, D), lambda g: (0, 0)),
            pl.BlockSpec((D, 2 * D), lambda g: (0, 0)),
            pl.BlockSpec((1, 2 * D), lambda g: (0, 0)),
        ],
        out_specs=pl.BlockSpec((NPOOL, 2 * D), lambda g: (0, 0)),
        out_shape=jax.ShapeDtypeStruct((NPOOL, 2 * D), BF16),
    )(pool_a, x2, w_ckv, b_ckv)

    rank = pl.pallas_call(
        _rank_kernel,
        grid=(S // QBLK,),
        in_specs=[
            pl.BlockSpec((QBLK, 1), lambda p: (p, 0)),
            pl.BlockSpec((1, S), lambda p: (0, 0)),
        ],
        out_specs=pl.BlockSpec((QBLK, 1), lambda p: (p, 0)),
        out_shape=jax.ShapeDtypeStruct((S, 1), F32),
    )(imp, imp.reshape(1, S))

    w_tkv = jnp.concatenate([W_tk, W_tv], axis=1)
    b_tkv = jnp.concatenate([b_tk, b_tv]).reshape(1, -1)
    tkv, tidx = pl.pallas_call(
        _sel_kernel,
        grid=(1,),
        in_specs=[
            pl.BlockSpec((1, S), lambda g: (0, 0)),
            pl.BlockSpec((S, D), lambda g: (0, 0)),
            pl.BlockSpec((D, 2 * D), lambda g: (0, 0)),
            pl.BlockSpec((1, 2 * D), lambda g: (0, 0)),
        ],
        out_specs=[
            pl.BlockSpec((TK, 2 * D), lambda g: (0, 0)),
            pl.BlockSpec((TK, 1), lambda g: (0, 0)),
        ],
        out_shape=[
            jax.ShapeDtypeStruct((TK, 2 * D), BF16),
            jax.ShapeDtypeStruct((TK, 1), F32),
        ],
    )(rank.reshape(1, S), x2, w_tkv, b_tkv)

    nhp = NH // 2
    local = pl.pallas_call(
        _local_attn_kernel,
        grid=(nhp, S // QBLK),
        in_specs=[
            pl.BlockSpec((QBLK, 128), lambda hp, i: (i, hp)),
            pl.BlockSpec((QBLK, 128), lambda hp, i: (jnp.maximum(i - 2, 0), 8 + hp)),
            pl.BlockSpec((QBLK, 128), lambda hp, i: (jnp.maximum(i - 1, 0), 8 + hp)),
            pl.BlockSpec((QBLK, 128), lambda hp, i: (i, 8 + hp)),
            pl.BlockSpec((QBLK, 128), lambda hp, i: (jnp.maximum(i - 2, 0), 16 + hp)),
            pl.BlockSpec((QBLK, 128), lambda hp, i: (jnp.maximum(i - 1, 0), 16 + hp)),
            pl.BlockSpec((QBLK, 128), lambda hp, i: (i, 16 + hp)),
        ],
        out_specs=pl.BlockSpec((QBLK, 128), lambda hp, i: (i, hp)),
        out_shape=jax.ShapeDtypeStruct((S, D), BF16),
    )(qkv, qkv, qkv, qkv, qkv, qkv, qkv)

    cout, tout = pl.pallas_call(
        _ctattn_kernel,
        grid=(nhp, S // CQBLK),
        in_specs=[
            pl.BlockSpec((CQBLK, 128), lambda hp, i: (i, hp)),
            pl.BlockSpec((CQBLK, 128), lambda hp, i: (i, hp)),
            pl.BlockSpec((NPOOL, 128), lambda hp, i: (0, hp)),
            pl.BlockSpec((NPOOL, 128), lambda hp, i: (0, 8 + hp)),
            pl.BlockSpec((TK, 128), lambda hp, i: (0, hp)),
            pl.BlockSpec((TK, 128), lambda hp, i: (0, 8 + hp)),
            pl.BlockSpec((1, TK), lambda hp, i: (0, 0)),
        ],
        out_specs=[
            pl.BlockSpec((CQBLK, 128), lambda hp, i: (i, hp)),
            pl.BlockSpec((CQBLK, 128), lambda hp, i: (i, hp)),
        ],
        out_shape=[
            jax.ShapeDtypeStruct((S, D), BF16),
            jax.ShapeDtypeStruct((S, D), BF16),
        ],
    )(cq, tq, ckcv, ckcv, tkv, tkv, tidx.reshape(1, TK))

    out = pl.pallas_call(
        _combine_kernel,
        grid=(S // PBLK,),
        in_specs=[
            pl.BlockSpec((PBLK, D), lambda m: (m, 0)),
            pl.BlockSpec((PBLK, D), lambda m: (m, 0)),
            pl.BlockSpec((PBLK, D), lambda m: (m, 0)),
            pl.BlockSpec((PBLK, D), lambda m: (m, 0)),
            pl.BlockSpec((D, D), lambda m: (0, 0)),
            pl.BlockSpec((1, D), lambda m: (0, 0)),
            pl.BlockSpec((D, D), lambda m: (0, 0)),
            pl.BlockSpec((1, D), lambda m: (0, 0)),
            pl.BlockSpec((D, D), lambda m: (0, 0)),
            pl.BlockSpec((1, D), lambda m: (0, 0)),
            pl.BlockSpec((D, D), lambda m: (0, 0)),
            pl.BlockSpec((1, D), lambda m: (0, 0)),
            pl.BlockSpec((D, D), lambda m: (0, 0)),
            pl.BlockSpec((1, D), lambda m: (0, 0)),
        ],
        out_specs=pl.BlockSpec((PBLK, D), lambda m: (m, 0)),
        out_shape=jax.ShapeDtypeStruct((S, D), F32),
    )(x2, local, cout, tout,
      W_lout, b_lout.reshape(1, -1), W_cout, b_cout.reshape(1, -1),
      W_tout, b_tout.reshape(1, -1), W_gc, b_gc.reshape(1, -1),
      W_gt, b_gt.reshape(1, -1))

    return out.reshape(1, S, D)
